# baseline (speedup 1.0000x reference)
import functools

import jax
import jax.numpy as jnp
from jax import lax
from jax.experimental import pallas as pl
from jax.experimental.pallas import tpu as pltpu
from jax.experimental.pallas import tpu_sc as plsc

BATCH = 16384
HIST = 50
EMBED = 64
B = BATCH * HIST
NODE = 1000000
NC = 2
NS = 16
NW = NC * NS
BPW = B // NW
NBUF = 4
CHUNK = 400
CBATCH = CHUNK // HIST
NCHUNK = BPW // CHUNK


@jax.jit
def _gather(idx, table):
    mesh = plsc.VectorSubcoreMesh(core_axis_name="c", subcore_axis_name="s")

    @functools.partial(
        pl.kernel,
        out_type=jax.ShapeDtypeStruct((BATCH, HIST, EMBED), jnp.float32),
        mesh=mesh,
        scratch_types=[
            pltpu.VMEM((BPW,), jnp.int32),
            [pltpu.VMEM((CHUNK, EMBED), jnp.float32) for _ in range(NBUF)],
            [pltpu.SemaphoreType.DMA for _ in range(NBUF)],
            [pltpu.SemaphoreType.DMA for _ in range(NBUF)],
        ],
        compiler_params=pltpu.CompilerParams(use_tc_tiling_on_sc=False),
    )
    def body(idx_hbm, table_hbm, out_hbm, idx_v, rows, gsem, wsem):
        wid = lax.axis_index("s") * NC + lax.axis_index("c")
        base = wid * BPW
        bbase = wid * (BPW // HIST)

        pltpu.sync_copy(idx_hbm.at[pl.ds(base, BPW)], idx_v)

        def gather_chunk(n, b):
            pltpu.async_copy(
                table_hbm.at[idx_v.at[pl.ds(n * CHUNK, CHUNK)]], rows[b],
                gsem[b])

        def write_chunk(g, b):
            for i in range(CBATCH):
                pltpu.async_copy(
                    rows[b].at[pl.ds(i * HIST, HIST), :],
                    out_hbm.at[bbase + g * CBATCH + i], wsem[b])

        def wait_write(g, b):
            for i in range(CBATCH):
                pltpu.make_async_copy(
                    rows[b].at[pl.ds(i * HIST, HIST), :],
                    out_hbm.at[bbase + g * CBATCH + i], wsem[b]).wait()

        gather_chunk(0, 0)

        def group(o):
            for b in range(NBUF):
                g = o * NBUF + b
                n = g + 1
                bn = (b + 1) % NBUF

                @pl.when(n < NCHUNK)
                def _():
                    @pl.when(n >= NBUF)
                    def _():
                        wait_write(n - NBUF, bn)
                    gather_chunk(n, bn)

                pltpu.make_async_copy(
                    table_hbm.at[idx_v.at[pl.ds(0, CHUNK)]], rows[b],
                    gsem[b]).wait()
                write_chunk(g, b)

        pl.loop(0, NCHUNK // NBUF)(group)

        for g in range(NCHUNK - NBUF, NCHUNK):
            wait_write(g, g % NBUF)

    return body(idx, table)


def kernel(x, m, table):
    del m
    idx = x.reshape(-1)
    return _gather(idx, table)
